# baseline (speedup 1.0000x reference)
import functools

import jax
import jax.numpy as jnp
from jax import lax
from jax.experimental import pallas as pl
from jax.experimental.pallas import tpu as pltpu
from jax.experimental.pallas import tpu_sc as plsc

NC = 2
NS = 16
LANES = 16

TRASH = 1024
C = 80


def _make_scatter_kernel(E, N, D):
    assert N % (2 * NC) == 0
    NH = N // NC
    AR = NH + TRASH
    assert E % NS == 0
    ES = E // NS
    assert ES % C == 0
    slab = (AR // NS) // 8 * 8
    last = AR - slab * (NS - 1)
    assert slab > 0 and last > 0 and AR % 8 == 0

    mesh = plsc.VectorSubcoreMesh(core_axis_name="c", subcore_axis_name="s")

    @functools.partial(
        pl.kernel,
        out_type=jax.ShapeDtypeStruct((NC, AR, D), jnp.float32),
        mesh=mesh,
        scratch_types=[
            pltpu.VMEM((C,), jnp.int32),
            pltpu.VMEM((C,), jnp.int32),
            pltpu.VMEM((C, D), jnp.float32),
            pltpu.VMEM((last, D), jnp.float32),
            pltpu.VMEM_SHARED((AR, D), jnp.float32),
            pltpu.SemaphoreType.DMA,
        ],
    )
    def scatter_kernel(src_hbm, dst_hbm, g_hbm, out_hbm,
                       sb_src, sb_dst, rows_v, zbuf_v,
                       acc_sh, sem):
        c = lax.axis_index("c")
        s = lax.axis_index("s")
        lo = c * NH
        hi = lo + NH

        def zrow(i, carry):
            for j in range(D // LANES):
                zbuf_v[i, pl.ds(j * LANES, LANES)] = jnp.zeros(
                    (LANES,), jnp.float32)
            return carry

        lax.fori_loop(0, last, zrow, 0)

        @pl.when(s < NS - 1)
        def _():
            pltpu.sync_copy(zbuf_v.at[pl.ds(0, slab)],
                            acc_sh.at[pl.ds(s * slab, slab)])

        @pl.when(s == NS - 1)
        def _():
            pltpu.sync_copy(zbuf_v, acc_sh.at[pl.ds((NS - 1) * slab, last)])

        plsc.subcore_barrier()

        base = s * ES

        def xfer(j, carry):
            off = base + j * C
            pltpu.sync_copy(src_hbm.at[pl.ds(off, C)], sb_src)
            pltpu.sync_copy(dst_hbm.at[pl.ds(off, C)], sb_dst)
            for k in range(C // LANES):
                d = sb_dst[pl.ds(k * LANES, LANES)]
                m = jnp.logical_and(d >= lo, d < hi)
                d_eff = jnp.where(m, d - lo, NH + (d & (TRASH - 1)))
                sb_dst[pl.ds(k * LANES, LANES)] = d_eff
            pltpu.async_copy(g_hbm.at[sb_src], rows_v, sem).wait()
            pltpu.sync_copy(rows_v, acc_sh.at[sb_dst], add=True)
            return carry

        lax.fori_loop(0, ES // C, xfer, 0)
        plsc.subcore_barrier()

        @pl.when(s < NS - 1)
        def _():
            pltpu.sync_copy(acc_sh.at[pl.ds(s * slab, slab)],
                            out_hbm.at[c, pl.ds(s * slab, slab)])

        @pl.when(s == NS - 1)
        def _():
            pltpu.sync_copy(acc_sh.at[pl.ds((NS - 1) * slab, last)],
                            out_hbm.at[c, pl.ds((NS - 1) * slab, last)])

    return scatter_kernel


def _tc1_body(x_ref, w1_ref, deg_ref, h1_ref, g1_ref, dis_ref):
    deg = deg_ref[:, 0:1] + 1.0
    dis = lax.rsqrt(deg)
    h1 = jnp.dot(x_ref[...], w1_ref[...], preferred_element_type=jnp.float32)
    h1_ref[...] = h1
    g1_ref[...] = h1 * dis
    dis_ref[...] = jnp.broadcast_to(dis, dis_ref.shape)


def _tc2_body(p_ref, h1_ref, dis_ref, b1_ref, w2_ref, h2_ref, g2_ref):
    dis = dis_ref[...]
    pre = dis * p_ref[...] + dis * dis * h1_ref[...] + b1_ref[...]
    t = jnp.maximum(pre, 0.0)
    h2 = jnp.dot(t, w2_ref[...], preferred_element_type=jnp.float32)
    h2_ref[...] = h2
    g2_ref[...] = h2 * dis


def _tc3_body(q_ref, h2_ref, dis_ref, b2_ref, out_ref):
    dis = dis_ref[...]
    out_ref[...] = dis * q_ref[...] + dis * dis * h2_ref[...] + b2_ref[...]


@jax.jit
def kernel(x, edge_index, W1, b1, W2, b2):
    N, D_in = x.shape
    D_hid = W1.shape[1]
    D_out = W2.shape[1]
    E = edge_index.shape[1]
    NH = N // NC
    f32 = jnp.float32

    ei = edge_index.astype(jnp.int32)
    src = ei[0]
    dst = ei[1]

    assert D_hid == D_out == D_in
    scat = _make_scatter_kernel(E, N, D_hid)

    def merge(o):
        return jnp.concatenate([o[0, :NH], o[1, :NH]], axis=0)

    ones_nd = jnp.ones((N, D_hid), f32)
    degc = merge(scat(src, dst, ones_nd))[:, :LANES]

    h1, g1, dis = pl.pallas_call(
        _tc1_body,
        out_shape=(
            jax.ShapeDtypeStruct((N, D_hid), f32),
            jax.ShapeDtypeStruct((N, D_hid), f32),
            jax.ShapeDtypeStruct((N, D_hid), f32),
        ),
    )(x, W1, degc)

    p = merge(scat(src, dst, g1))

    h2, g2 = pl.pallas_call(
        _tc2_body,
        out_shape=(
            jax.ShapeDtypeStruct((N, D_out), f32),
            jax.ShapeDtypeStruct((N, D_out), f32),
        ),
    )(p, h1, dis, b1.reshape(1, -1), W2)

    q = merge(scat(src, dst, g2))

    out = pl.pallas_call(
        _tc3_body,
        out_shape=jax.ShapeDtypeStruct((N, D_out), f32),
    )(q, h2, dis, b2.reshape(1, -1))

    return out


# speedup vs baseline: 2.3535x; 2.3535x over previous
import functools

import jax
import jax.numpy as jnp
from jax import lax
from jax.experimental import pallas as pl
from jax.experimental.pallas import tpu as pltpu
from jax.experimental.pallas import tpu_sc as plsc

NC = 2
NS = 16
LANES = 16

TRASH = 1024
C = 80


def _make_scatter_kernel(E, N, D):
    assert N % (2 * NC) == 0
    NH = N // NC
    AR = NH + TRASH
    assert E % NS == 0
    ES = E // NS
    assert ES % C == 0
    SR = 25
    assert (ES // C) % SR == 0
    slab = (AR // NS) // 8 * 8
    last = AR - slab * (NS - 1)
    assert slab > 0 and last > 0 and AR % 8 == 0

    mesh = plsc.VectorSubcoreMesh(core_axis_name="c", subcore_axis_name="s")

    @functools.partial(
        pl.kernel,
        out_type=jax.ShapeDtypeStruct((NC, AR, D), jnp.float32),
        mesh=mesh,
        scratch_types=[
            pltpu.VMEM((SR, C), jnp.int32),
            pltpu.VMEM((SR, C), jnp.int32),
            pltpu.VMEM((C, D), jnp.float32),
            pltpu.VMEM((C, D), jnp.float32),
            pltpu.VMEM((last, D), jnp.float32),
            pltpu.VMEM_SHARED((AR, D), jnp.float32),
            pltpu.SemaphoreType.DMA,
            pltpu.SemaphoreType.DMA,
        ],
    )
    def scatter_kernel(src_hbm, dst_hbm, g_hbm, out_hbm,
                       sb_src, sb_dst, rows_a, rows_b, zbuf_v,
                       acc_sh, sem_a, sem_b):
        c = lax.axis_index("c")
        s = lax.axis_index("s")
        lo = c * NH
        hi = lo + NH

        def zrow(i, carry):
            for j in range(D // LANES):
                zbuf_v[i, pl.ds(j * LANES, LANES)] = jnp.zeros(
                    (LANES,), jnp.float32)
            return carry

        lax.fori_loop(0, last, zrow, 0)

        @pl.when(s < NS - 1)
        def _():
            pltpu.sync_copy(zbuf_v.at[pl.ds(0, slab)],
                            acc_sh.at[pl.ds(s * slab, slab)])

        @pl.when(s == NS - 1)
        def _():
            pltpu.sync_copy(zbuf_v, acc_sh.at[pl.ds((NS - 1) * slab, last)])

        plsc.subcore_barrier()

        rows_bufs = (rows_a, rows_b)
        sems = (sem_a, sem_b)

        def stage(t, carry):
            pltpu.sync_copy(src_hbm.at[s, t], sb_src)
            pltpu.sync_copy(dst_hbm.at[s, t], sb_dst)
            for r in range(SR):
                for k in range(C // LANES):
                    d = sb_dst[r, pl.ds(k * LANES, LANES)]
                    m = jnp.logical_and(d >= lo, d < hi)
                    d_eff = jnp.where(m, d - lo, NH + (d & (TRASH - 1)))
                    sb_dst[r, pl.ds(k * LANES, LANES)] = d_eff
            descs = [None, None]
            descs[0] = pltpu.async_copy(
                g_hbm.at[sb_src.at[0]], rows_bufs[0], sems[0])
            for j in range(SR):
                if j + 1 < SR:
                    descs[(j + 1) % 2] = pltpu.async_copy(
                        g_hbm.at[sb_src.at[j + 1]],
                        rows_bufs[(j + 1) % 2], sems[(j + 1) % 2])
                descs[j % 2].wait()
                pltpu.sync_copy(
                    rows_bufs[j % 2], acc_sh.at[sb_dst.at[j]], add=True)
            return carry

        lax.fori_loop(0, (ES // C) // SR, stage, 0)
        plsc.subcore_barrier()

        @pl.when(s < NS - 1)
        def _():
            pltpu.sync_copy(acc_sh.at[pl.ds(s * slab, slab)],
                            out_hbm.at[c, pl.ds(s * slab, slab)])

        @pl.when(s == NS - 1)
        def _():
            pltpu.sync_copy(acc_sh.at[pl.ds((NS - 1) * slab, last)],
                            out_hbm.at[c, pl.ds((NS - 1) * slab, last)])

    return scatter_kernel


def _tc1_body(x_ref, w1_ref, deg_ref, h1_ref, g1_ref, dis_ref):
    deg = deg_ref[:, 0:1] + 1.0
    dis = lax.rsqrt(deg)
    h1 = jnp.dot(x_ref[...], w1_ref[...], preferred_element_type=jnp.float32)
    h1_ref[...] = h1
    g1_ref[...] = h1 * dis
    dis_ref[...] = jnp.broadcast_to(dis, dis_ref.shape)


def _tc2_body(p_ref, h1_ref, dis_ref, b1_ref, w2_ref, h2_ref, g2_ref):
    dis = dis_ref[...]
    pre = dis * p_ref[...] + dis * dis * h1_ref[...] + b1_ref[...]
    t = jnp.maximum(pre, 0.0)
    h2 = jnp.dot(t, w2_ref[...], preferred_element_type=jnp.float32)
    h2_ref[...] = h2
    g2_ref[...] = h2 * dis


def _tc3_body(q_ref, h2_ref, dis_ref, b2_ref, out_ref):
    dis = dis_ref[...]
    out_ref[...] = dis * q_ref[...] + dis * dis * h2_ref[...] + b2_ref[...]


@jax.jit
def kernel(x, edge_index, W1, b1, W2, b2):
    N, D_in = x.shape
    D_hid = W1.shape[1]
    D_out = W2.shape[1]
    E = edge_index.shape[1]
    NH = N // NC
    f32 = jnp.float32

    ei = edge_index.astype(jnp.int32)
    stages = E // C // 25 // NS
    src = ei[0].reshape(NS, stages, 25, C)
    dst = ei[1].reshape(NS, stages, 25, C)

    assert D_hid == D_out == D_in
    scat = _make_scatter_kernel(E, N, D_hid)

    def merge(o):
        return jnp.concatenate([o[0, :NH], o[1, :NH]], axis=0)

    ones_nd = jnp.ones((N, D_hid), f32)
    degc = merge(scat(src, dst, ones_nd))[:, :LANES]

    h1, g1, dis = pl.pallas_call(
        _tc1_body,
        out_shape=(
            jax.ShapeDtypeStruct((N, D_hid), f32),
            jax.ShapeDtypeStruct((N, D_hid), f32),
            jax.ShapeDtypeStruct((N, D_hid), f32),
        ),
    )(x, W1, degc)

    p = merge(scat(src, dst, g1))

    h2, g2 = pl.pallas_call(
        _tc2_body,
        out_shape=(
            jax.ShapeDtypeStruct((N, D_out), f32),
            jax.ShapeDtypeStruct((N, D_out), f32),
        ),
    )(p, h1, dis, b1.reshape(1, -1), W2)

    q = merge(scat(src, dst, g2))

    out = pl.pallas_call(
        _tc3_body,
        out_shape=jax.ShapeDtypeStruct((N, D_out), f32),
    )(q, h2, dis, b2.reshape(1, -1))

    return out


# speedup vs baseline: 2.8483x; 1.2102x over previous
import functools

import jax
import jax.numpy as jnp
from jax import lax
from jax.experimental import pallas as pl
from jax.experimental.pallas import tpu as pltpu
from jax.experimental.pallas import tpu_sc as plsc

NC = 2
NS = 16
LANES = 16

TRASH = 1024
C = 80


def _make_scatter_kernel(E, N, D):
    DH = D // NC
    assert D % NC == 0
    assert E % NS == 0
    ES = E // NS
    assert ES % C == 0
    SR = 25
    assert (ES // C) % SR == 0
    slab = -(-N // (NS * 8)) * 8
    AR = slab * NS

    mesh = plsc.VectorSubcoreMesh(core_axis_name="c", subcore_axis_name="s")

    @functools.partial(
        pl.kernel,
        out_type=jax.ShapeDtypeStruct((NC, AR, DH), jnp.float32),
        mesh=mesh,
        compiler_params=pltpu.CompilerParams(use_tc_tiling_on_sc=False),
        scratch_types=[
            pltpu.VMEM((SR, C), jnp.int32),
            pltpu.VMEM((SR, C), jnp.int32),
            pltpu.VMEM((C, DH), jnp.float32),
            pltpu.VMEM((C, DH), jnp.float32),
            pltpu.VMEM((slab, DH), jnp.float32),
            pltpu.VMEM_SHARED((AR, DH), jnp.float32),
            pltpu.SemaphoreType.DMA,
            pltpu.SemaphoreType.DMA,
        ],
    )
    def scatter_kernel(src_hbm, dst_hbm, g_hbm, out_hbm,
                       sb_src, sb_dst, rows_a, rows_b, zbuf_v,
                       acc_sh, sem_a, sem_b):
        c = lax.axis_index("c")
        s = lax.axis_index("s")

        def zrow(i, carry):
            for j in range(DH // LANES):
                zbuf_v[i, pl.ds(j * LANES, LANES)] = jnp.zeros(
                    (LANES,), jnp.float32)
            return carry

        lax.fori_loop(0, slab, zrow, 0)
        pltpu.sync_copy(zbuf_v, acc_sh.at[pl.ds(s * slab, slab)])
        plsc.subcore_barrier()

        rows_bufs = (rows_a, rows_b)
        sems = (sem_a, sem_b)

        def stage(t, carry):
            pltpu.sync_copy(src_hbm.at[c, s, t], sb_src)
            pltpu.sync_copy(dst_hbm.at[s, t], sb_dst)
            descs = [None, None]
            descs[0] = pltpu.async_copy(
                g_hbm.at[sb_src.at[0]], rows_bufs[0], sems[0])
            for j in range(SR):
                if j + 1 < SR:
                    descs[(j + 1) % 2] = pltpu.async_copy(
                        g_hbm.at[sb_src.at[j + 1]],
                        rows_bufs[(j + 1) % 2], sems[(j + 1) % 2])
                descs[j % 2].wait()
                pltpu.sync_copy(
                    rows_bufs[j % 2], acc_sh.at[sb_dst.at[j]], add=True)
            return carry

        lax.fori_loop(0, (ES // C) // SR, stage, 0)
        plsc.subcore_barrier()

        pltpu.sync_copy(acc_sh.at[pl.ds(s * slab, slab)],
                        out_hbm.at[c, pl.ds(s * slab, slab)])

    return scatter_kernel


def _tc1_body(x_ref, w1_ref, deg_ref, h1_ref, g1_ref, dis_ref):
    deg = deg_ref[:, 0:1] + 1.0
    dis = lax.rsqrt(deg)
    h1 = jnp.dot(x_ref[...], w1_ref[...], preferred_element_type=jnp.float32)
    h1_ref[...] = h1
    g1_ref[...] = h1 * dis
    dis_ref[...] = jnp.broadcast_to(dis, dis_ref.shape)


def _tc2_body(p_ref, h1_ref, dis_ref, b1_ref, w2_ref, h2_ref, g2_ref):
    dis = dis_ref[...]
    pre = dis * p_ref[...] + dis * dis * h1_ref[...] + b1_ref[...]
    t = jnp.maximum(pre, 0.0)
    h2 = jnp.dot(t, w2_ref[...], preferred_element_type=jnp.float32)
    h2_ref[...] = h2
    g2_ref[...] = h2 * dis


def _tc3_body(q_ref, h2_ref, dis_ref, b2_ref, out_ref):
    dis = dis_ref[...]
    out_ref[...] = dis * q_ref[...] + dis * dis * h2_ref[...] + b2_ref[...]


@jax.jit
def kernel(x, edge_index, W1, b1, W2, b2):
    N, D_in = x.shape
    D_hid = W1.shape[1]
    D_out = W2.shape[1]
    E = edge_index.shape[1]
    NH = N // NC
    f32 = jnp.float32

    ei = edge_index.astype(jnp.int32)
    stages = E // C // 25 // NS
    src0 = ei[0].reshape(NS, stages, 25, C)
    src = jnp.stack([src0, src0 + N])
    dst = ei[1].reshape(NS, stages, 25, C)

    assert D_hid == D_out == D_in
    DH = D_hid // NC
    scat = _make_scatter_kernel(E, N, D_hid)

    def split_cols(g):
        return jnp.concatenate([g[:, :DH], g[:, DH:]], axis=0)

    def merge(o):
        return jnp.concatenate([o[0, :N], o[1, :N]], axis=1)

    ones_nd = jnp.ones((NC * N, DH), f32)
    degc = scat(src, dst, ones_nd)[0, :N, :LANES]

    h1, g1, dis = pl.pallas_call(
        _tc1_body,
        out_shape=(
            jax.ShapeDtypeStruct((N, D_hid), f32),
            jax.ShapeDtypeStruct((N, D_hid), f32),
            jax.ShapeDtypeStruct((N, D_hid), f32),
        ),
    )(x, W1, degc)

    p = merge(scat(src, dst, split_cols(g1)))

    h2, g2 = pl.pallas_call(
        _tc2_body,
        out_shape=(
            jax.ShapeDtypeStruct((N, D_out), f32),
            jax.ShapeDtypeStruct((N, D_out), f32),
        ),
    )(p, h1, dis, b1.reshape(1, -1), W2)

    q = merge(scat(src, dst, split_cols(g2)))

    out = pl.pallas_call(
        _tc3_body,
        out_shape=jax.ShapeDtypeStruct((N, D_out), f32),
    )(q, h2, dis, b2.reshape(1, -1))

    return out


# speedup vs baseline: 3.7625x; 1.3210x over previous
import functools

import jax
import jax.numpy as jnp
from jax import lax
from jax.experimental import pallas as pl
from jax.experimental.pallas import tpu as pltpu
from jax.experimental.pallas import tpu_sc as plsc

NC = 2
NS = 16
LANES = 16

TRASH = 1024
C = 80


def _make_scatter_kernel(E, N, D):
    DH = D // NC
    assert D % NC == 0
    assert E % NS == 0
    ES = E // NS
    assert ES % C == 0
    SR = 25
    assert (ES // C) % SR == 0
    slab = -(-N // (NS * 8)) * 8
    AR = slab * NS

    mesh = plsc.VectorSubcoreMesh(core_axis_name="c", subcore_axis_name="s")

    @functools.partial(
        pl.kernel,
        out_type=jax.ShapeDtypeStruct((NC, AR, DH), jnp.float32),
        mesh=mesh,
        compiler_params=pltpu.CompilerParams(use_tc_tiling_on_sc=False),
        scratch_types=[
            pltpu.VMEM((SR, C), jnp.int32),
            pltpu.VMEM((SR, C), jnp.int32),
            pltpu.VMEM((C, DH), jnp.float32),
            pltpu.VMEM((C, DH), jnp.float32),
            pltpu.VMEM((slab, DH), jnp.float32),
            pltpu.VMEM_SHARED((AR, DH), jnp.float32),
            pltpu.SemaphoreType.DMA,
            pltpu.SemaphoreType.DMA,
        ],
    )
    def scatter_kernel(src_hbm, dst_hbm, g_hbm, out_hbm,
                       sb_src, sb_dst, rows_a, rows_b, zbuf_v,
                       acc_sh, sem_a, sem_b):
        c = lax.axis_index("c")
        s = lax.axis_index("s")

        def zrow(i, carry):
            for j in range(DH // LANES):
                zbuf_v[i, pl.ds(j * LANES, LANES)] = jnp.zeros(
                    (LANES,), jnp.float32)
            return carry

        lax.fori_loop(0, slab, zrow, 0)
        pltpu.sync_copy(zbuf_v, acc_sh.at[pl.ds(s * slab, slab)])
        plsc.subcore_barrier()

        rows_bufs = (rows_a, rows_b)
        sems = (sem_a, sem_b)

        def stage(t, carry):
            pltpu.sync_copy(src_hbm.at[c, s, t], sb_src)
            pltpu.sync_copy(dst_hbm.at[s, t], sb_dst)
            descs = [None, None]
            descs[0] = pltpu.async_copy(
                g_hbm.at[sb_src.at[0]], rows_bufs[0], sems[0])
            for j in range(SR):
                if j + 1 < SR:
                    descs[(j + 1) % 2] = pltpu.async_copy(
                        g_hbm.at[sb_src.at[j + 1]],
                        rows_bufs[(j + 1) % 2], sems[(j + 1) % 2])
                descs[j % 2].wait()
                pltpu.sync_copy(
                    rows_bufs[j % 2], acc_sh.at[sb_dst.at[j]], add=True)
            return carry

        lax.fori_loop(0, (ES // C) // SR, stage, 0)
        plsc.subcore_barrier()

        pltpu.sync_copy(acc_sh.at[pl.ds(s * slab, slab)],
                        out_hbm.at[c, pl.ds(s * slab, slab)])

    return scatter_kernel


def _make_count_kernel(E, N):
    assert E % NS == 0
    ES = E // NS
    assert ES % C == 0
    SR = 25
    stages = (ES // C) // SR
    assert stages % NC == 0
    slab = -(-N // (NS * 8)) * 8
    AR = slab * NS

    mesh = plsc.VectorSubcoreMesh(core_axis_name="c", subcore_axis_name="s")

    @functools.partial(
        pl.kernel,
        out_type=jax.ShapeDtypeStruct((NC, AR, LANES), jnp.float32),
        mesh=mesh,
        compiler_params=pltpu.CompilerParams(use_tc_tiling_on_sc=False),
        scratch_types=[
            pltpu.VMEM((SR, C), jnp.int32),
            pltpu.VMEM((C, LANES), jnp.float32),
            pltpu.VMEM((slab, LANES), jnp.float32),
            pltpu.VMEM_SHARED((AR, LANES), jnp.float32),
            pltpu.SemaphoreType.DMA,
        ],
    )
    def count_kernel(dst_hbm, out_hbm, sb_dst, ones_v, zbuf_v, acc_sh, sem):
        c = lax.axis_index("c")
        s = lax.axis_index("s")

        def orow(i, carry):
            ones_v[i, :] = jnp.full((LANES,), 1.0, jnp.float32)
            zbuf_v[i, :] = jnp.zeros((LANES,), jnp.float32)
            return carry

        lax.fori_loop(0, C, orow, 0)

        def zrow(i, carry):
            zbuf_v[i, :] = jnp.zeros((LANES,), jnp.float32)
            return carry

        lax.fori_loop(0, slab, zrow, 0)
        pltpu.sync_copy(zbuf_v, acc_sh.at[pl.ds(s * slab, slab)])
        plsc.subcore_barrier()

        def stage(t, carry):
            pltpu.sync_copy(
                dst_hbm.at[s, c * (stages // NC) + t], sb_dst)
            for j in range(SR):
                pltpu.async_copy(
                    ones_v, acc_sh.at[sb_dst.at[j]], sem, add=True)
            for j in range(SR):
                pltpu.make_async_copy(
                    ones_v, acc_sh.at[sb_dst.at[j]], sem).wait()
            return carry

        lax.fori_loop(0, stages // NC, stage, 0)
        plsc.subcore_barrier()
        pltpu.sync_copy(acc_sh.at[pl.ds(s * slab, slab)],
                        out_hbm.at[c, pl.ds(s * slab, slab)])

    return count_kernel


def _tc1_body(x_ref, w1_ref, deg_ref, h1_ref, g1_ref, dis_ref):
    deg = deg_ref[:, 0:1] + 1.0
    dis = lax.rsqrt(deg)
    h1 = jnp.dot(x_ref[...], w1_ref[...], preferred_element_type=jnp.float32)
    h1_ref[...] = h1
    g1_ref[...] = h1 * dis
    dis_ref[...] = jnp.broadcast_to(dis, dis_ref.shape)


def _tc2_body(p_ref, h1_ref, dis_ref, b1_ref, w2_ref, h2_ref, g2_ref):
    dis = dis_ref[...]
    pre = dis * p_ref[...] + dis * dis * h1_ref[...] + b1_ref[...]
    t = jnp.maximum(pre, 0.0)
    h2 = jnp.dot(t, w2_ref[...], preferred_element_type=jnp.float32)
    h2_ref[...] = h2
    g2_ref[...] = h2 * dis


def _tc3_body(q_ref, h2_ref, dis_ref, b2_ref, out_ref):
    dis = dis_ref[...]
    out_ref[...] = dis * q_ref[...] + dis * dis * h2_ref[...] + b2_ref[...]


@jax.jit
def kernel(x, edge_index, W1, b1, W2, b2):
    N, D_in = x.shape
    D_hid = W1.shape[1]
    D_out = W2.shape[1]
    E = edge_index.shape[1]
    NH = N // NC
    f32 = jnp.float32

    ei = edge_index.astype(jnp.int32)
    stages = E // C // 25 // NS
    src0 = ei[0].reshape(NS, stages, 25, C)
    src = jnp.stack([src0, src0 + N])
    dst = ei[1].reshape(NS, stages, 25, C)

    assert D_hid == D_out == D_in
    DH = D_hid // NC
    scat = _make_scatter_kernel(E, N, D_hid)

    def split_cols(g):
        return jnp.concatenate([g[:, :DH], g[:, DH:]], axis=0)

    def merge(o):
        return jnp.concatenate([o[0, :N], o[1, :N]], axis=1)

    cnt = _make_count_kernel(E, N)(dst)
    degc = cnt[0, :N] + cnt[1, :N]

    h1, g1, dis = pl.pallas_call(
        _tc1_body,
        out_shape=(
            jax.ShapeDtypeStruct((N, D_hid), f32),
            jax.ShapeDtypeStruct((N, D_hid), f32),
            jax.ShapeDtypeStruct((N, D_hid), f32),
        ),
    )(x, W1, degc)

    p = merge(scat(src, dst, split_cols(g1)))

    h2, g2 = pl.pallas_call(
        _tc2_body,
        out_shape=(
            jax.ShapeDtypeStruct((N, D_out), f32),
            jax.ShapeDtypeStruct((N, D_out), f32),
        ),
    )(p, h1, dis, b1.reshape(1, -1), W2)

    q = merge(scat(src, dst, split_cols(g2)))

    out = pl.pallas_call(
        _tc3_body,
        out_shape=jax.ShapeDtypeStruct((N, D_out), f32),
    )(q, h2, dis, b2.reshape(1, -1))

    return out


# speedup vs baseline: 3.7672x; 1.0013x over previous
import functools

import jax
import jax.numpy as jnp
from jax import lax
from jax.experimental import pallas as pl
from jax.experimental.pallas import tpu as pltpu
from jax.experimental.pallas import tpu_sc as plsc

NC = 2
NS = 16
LANES = 16

TRASH = 1024
C = 80


def _make_scatter_kernel(E, N, D):
    DH = D // NC
    assert D % NC == 0
    assert E % NS == 0
    ES = E // NS
    assert ES % C == 0
    SR = 25
    assert (ES // C) % SR == 0
    slab = -(-N // (NS * 8)) * 8
    AR = slab * NS

    mesh = plsc.VectorSubcoreMesh(core_axis_name="c", subcore_axis_name="s")

    @functools.partial(
        pl.kernel,
        out_type=jax.ShapeDtypeStruct((NC, AR, DH), jnp.float32),
        mesh=mesh,
        compiler_params=pltpu.CompilerParams(use_tc_tiling_on_sc=False),
        scratch_types=[
            pltpu.VMEM((SR, C), jnp.int32),
            pltpu.VMEM((SR, C), jnp.int32),
            pltpu.VMEM((C, DH), jnp.float32),
            pltpu.VMEM((C, DH), jnp.float32),
            pltpu.VMEM((slab, DH), jnp.float32),
            pltpu.VMEM_SHARED((AR, DH), jnp.float32),
            pltpu.SemaphoreType.DMA,
            pltpu.SemaphoreType.DMA,
            pltpu.SemaphoreType.DMA,
            pltpu.SemaphoreType.DMA,
        ],
    )
    def scatter_kernel(src_hbm, dst_hbm, g_hbm, out_hbm,
                       sb_src, sb_dst, rows_a, rows_b, zbuf_v,
                       acc_sh, sem_a, sem_b, sem_sa, sem_sb):
        c = lax.axis_index("c")
        s = lax.axis_index("s")

        def zrow(i, carry):
            for j in range(DH // LANES):
                zbuf_v[i, pl.ds(j * LANES, LANES)] = jnp.zeros(
                    (LANES,), jnp.float32)
            return carry

        lax.fori_loop(0, slab, zrow, 0)
        pltpu.sync_copy(zbuf_v, acc_sh.at[pl.ds(s * slab, slab)])
        plsc.subcore_barrier()

        rows_bufs = (rows_a, rows_b)
        sems = (sem_a, sem_b)
        ssems = (sem_sa, sem_sb)

        def stage(t, carry):
            pltpu.sync_copy(src_hbm.at[c, s, t], sb_src)
            pltpu.sync_copy(dst_hbm.at[s, t], sb_dst)
            gd = [None, None]
            gd[0] = pltpu.async_copy(
                g_hbm.at[sb_src.at[0]], rows_bufs[0], sems[0])
            for j in range(SR):
                b = j % 2
                if j >= 1:
                    pltpu.make_async_copy(
                        rows_bufs[1 - b], acc_sh.at[sb_dst.at[j - 1]],
                        ssems[1 - b]).wait()
                if j + 1 < SR:
                    gd[1 - b] = pltpu.async_copy(
                        g_hbm.at[sb_src.at[j + 1]],
                        rows_bufs[1 - b], sems[1 - b])
                gd[b].wait()
                pltpu.async_copy(
                    rows_bufs[b], acc_sh.at[sb_dst.at[j]], ssems[b],
                    add=True)
            pltpu.make_async_copy(
                rows_bufs[(SR - 1) % 2], acc_sh.at[sb_dst.at[SR - 1]],
                ssems[(SR - 1) % 2]).wait()
            return carry

        lax.fori_loop(0, (ES // C) // SR, stage, 0)
        plsc.subcore_barrier()

        pltpu.sync_copy(acc_sh.at[pl.ds(s * slab, slab)],
                        out_hbm.at[c, pl.ds(s * slab, slab)])

    return scatter_kernel


def _make_count_kernel(E, N):
    assert E % NS == 0
    ES = E // NS
    assert ES % C == 0
    SR = 25
    stages = (ES // C) // SR
    assert stages % NC == 0
    slab = -(-N // (NS * 8)) * 8
    AR = slab * NS

    mesh = plsc.VectorSubcoreMesh(core_axis_name="c", subcore_axis_name="s")

    @functools.partial(
        pl.kernel,
        out_type=jax.ShapeDtypeStruct((NC, AR, LANES), jnp.float32),
        mesh=mesh,
        compiler_params=pltpu.CompilerParams(use_tc_tiling_on_sc=False),
        scratch_types=[
            pltpu.VMEM((SR, C), jnp.int32),
            pltpu.VMEM((C, LANES), jnp.float32),
            pltpu.VMEM((slab, LANES), jnp.float32),
            pltpu.VMEM_SHARED((AR, LANES), jnp.float32),
            pltpu.SemaphoreType.DMA,
        ],
    )
    def count_kernel(dst_hbm, out_hbm, sb_dst, ones_v, zbuf_v, acc_sh, sem):
        c = lax.axis_index("c")
        s = lax.axis_index("s")

        def orow(i, carry):
            ones_v[i, :] = jnp.full((LANES,), 1.0, jnp.float32)
            zbuf_v[i, :] = jnp.zeros((LANES,), jnp.float32)
            return carry

        lax.fori_loop(0, C, orow, 0)

        def zrow(i, carry):
            zbuf_v[i, :] = jnp.zeros((LANES,), jnp.float32)
            return carry

        lax.fori_loop(0, slab, zrow, 0)
        pltpu.sync_copy(zbuf_v, acc_sh.at[pl.ds(s * slab, slab)])
        plsc.subcore_barrier()

        def stage(t, carry):
            pltpu.sync_copy(
                dst_hbm.at[s, c * (stages // NC) + t], sb_dst)
            for j in range(SR):
                pltpu.async_copy(
                    ones_v, acc_sh.at[sb_dst.at[j]], sem, add=True)
            for j in range(SR):
                pltpu.make_async_copy(
                    ones_v, acc_sh.at[sb_dst.at[j]], sem).wait()
            return carry

        lax.fori_loop(0, stages // NC, stage, 0)
        plsc.subcore_barrier()
        pltpu.sync_copy(acc_sh.at[pl.ds(s * slab, slab)],
                        out_hbm.at[c, pl.ds(s * slab, slab)])

    return count_kernel


def _tc1_body(x_ref, w1_ref, deg_ref, h1_ref, g1_ref, dis_ref):
    deg = deg_ref[:, 0:1] + 1.0
    dis = lax.rsqrt(deg)
    h1 = jnp.dot(x_ref[...], w1_ref[...], preferred_element_type=jnp.float32)
    h1_ref[...] = h1
    g1_ref[...] = h1 * dis
    dis_ref[...] = jnp.broadcast_to(dis, dis_ref.shape)


def _tc2_body(p_ref, h1_ref, dis_ref, b1_ref, w2_ref, h2_ref, g2_ref):
    dis = dis_ref[...]
    pre = dis * p_ref[...] + dis * dis * h1_ref[...] + b1_ref[...]
    t = jnp.maximum(pre, 0.0)
    h2 = jnp.dot(t, w2_ref[...], preferred_element_type=jnp.float32)
    h2_ref[...] = h2
    g2_ref[...] = h2 * dis


def _tc3_body(q_ref, h2_ref, dis_ref, b2_ref, out_ref):
    dis = dis_ref[...]
    out_ref[...] = dis * q_ref[...] + dis * dis * h2_ref[...] + b2_ref[...]


@jax.jit
def kernel(x, edge_index, W1, b1, W2, b2):
    N, D_in = x.shape
    D_hid = W1.shape[1]
    D_out = W2.shape[1]
    E = edge_index.shape[1]
    NH = N // NC
    f32 = jnp.float32

    ei = edge_index.astype(jnp.int32)
    stages = E // C // 25 // NS
    src0 = ei[0].reshape(NS, stages, 25, C)
    src = jnp.stack([src0, src0 + N])
    dst = ei[1].reshape(NS, stages, 25, C)

    assert D_hid == D_out == D_in
    DH = D_hid // NC
    scat = _make_scatter_kernel(E, N, D_hid)

    def split_cols(g):
        return jnp.concatenate([g[:, :DH], g[:, DH:]], axis=0)

    def merge(o):
        return jnp.concatenate([o[0, :N], o[1, :N]], axis=1)

    cnt = _make_count_kernel(E, N)(dst)
    degc = cnt[0, :N] + cnt[1, :N]

    h1, g1, dis = pl.pallas_call(
        _tc1_body,
        out_shape=(
            jax.ShapeDtypeStruct((N, D_hid), f32),
            jax.ShapeDtypeStruct((N, D_hid), f32),
            jax.ShapeDtypeStruct((N, D_hid), f32),
        ),
    )(x, W1, degc)

    p = merge(scat(src, dst, split_cols(g1)))

    h2, g2 = pl.pallas_call(
        _tc2_body,
        out_shape=(
            jax.ShapeDtypeStruct((N, D_out), f32),
            jax.ShapeDtypeStruct((N, D_out), f32),
        ),
    )(p, h1, dis, b1.reshape(1, -1), W2)

    q = merge(scat(src, dst, split_cols(g2)))

    out = pl.pallas_call(
        _tc3_body,
        out_shape=jax.ShapeDtypeStruct((N, D_out), f32),
    )(q, h2, dis, b2.reshape(1, -1))

    return out
